# baseline (device time: 133117 ns/iter reference)
import jax
import jax.numpy as jnp
from jax import lax
from jax.experimental import pallas as pl
from jax.experimental.pallas import tpu as pltpu

N_DEV = 32


def kernel(t, W):
    m, k = t.shape
    n = W.shape[1]
    ch = m // N_DEV

    def body(t_ref, w_ref, out_ref, comm_ref, rs_send, rs_recv, ag_send, ag_recv):
        my = lax.axis_index("i")
        left = jnp.mod(my - 1, N_DEV)
        right = jnp.mod(my + 1, N_DEV)

        barrier = pltpu.get_barrier_semaphore()
        for nbr in (left, right):
            pl.semaphore_signal(
                barrier, inc=1,
                device_id=(nbr,), device_id_type=pl.DeviceIdType.MESH,
            )
        pl.semaphore_wait(barrier, 2)

        comm_ref[N_DEV - 1, :, :] = t_ref[pl.ds(my * ch, ch), :]
        for s in range(N_DEV - 1):
            src_slot = (N_DEV - 1) if s == 0 else (s - 1)
            rdma = pltpu.make_async_remote_copy(
                src_ref=comm_ref.at[src_slot],
                dst_ref=comm_ref.at[s],
                send_sem=rs_send.at[s],
                recv_sem=rs_recv.at[s],
                device_id=(right,),
                device_id_type=pl.DeviceIdType.MESH,
            )
            rdma.start()
            rdma.wait()
            c_recv = jnp.mod(my - s - 1, N_DEV)
            comm_ref[s, :, :] = comm_ref[s, :, :] + t_ref[pl.ds(c_recv * ch, ch), :]

        own = jnp.mod(my + 1, N_DEV)
        y = jnp.dot(
            comm_ref[N_DEV - 2, :, :], w_ref[:, :],
            preferred_element_type=jnp.float32,
        )
        out_ref[pl.ds(own * ch, ch), :] = y

        for s in range(N_DEV - 1):
            c_send = jnp.mod(my + 1 - s, N_DEV)
            sl = pl.ds(c_send * ch, ch)
            rdma = pltpu.make_async_remote_copy(
                src_ref=out_ref.at[sl],
                dst_ref=out_ref.at[sl],
                send_sem=ag_send.at[s],
                recv_sem=ag_recv.at[s],
                device_id=(right,),
                device_id_type=pl.DeviceIdType.MESH,
            )
            rdma.start()
            rdma.wait()

    return pl.pallas_call(
        body,
        out_shape=jax.ShapeDtypeStruct((m, n), jnp.float32),
        in_specs=[
            pl.BlockSpec(memory_space=pltpu.VMEM),
            pl.BlockSpec(memory_space=pltpu.VMEM),
        ],
        out_specs=pl.BlockSpec(memory_space=pltpu.VMEM),
        scratch_shapes=[
            pltpu.VMEM((N_DEV, ch, k), jnp.float32),
            pltpu.SemaphoreType.DMA((N_DEV - 1,)),
            pltpu.SemaphoreType.DMA((N_DEV - 1,)),
            pltpu.SemaphoreType.DMA((N_DEV - 1,)),
            pltpu.SemaphoreType.DMA((N_DEV - 1,)),
        ],
        compiler_params=pltpu.CompilerParams(collective_id=0),
    )(t, W)


# device time: 27472 ns/iter; 4.8456x vs baseline; 4.8456x over previous
import jax
import jax.numpy as jnp
from jax import lax
from jax.experimental import pallas as pl
from jax.experimental.pallas import tpu as pltpu

N_DEV = 32


def kernel(t, W):
    m, k = t.shape
    n = W.shape[1]
    ch = m // N_DEV

    def body(t_ref, w_ref, out_ref, comm_ref,
             rs_ssem, rs_rsem, ag_ssem, ag_rsem):
        my = lax.axis_index("i")

        barrier = pltpu.get_barrier_semaphore()
        for kk in range(1, N_DEV):
            peer = jnp.mod(my + kk, N_DEV)
            pl.semaphore_signal(
                barrier, inc=1,
                device_id=(peer,), device_id_type=pl.DeviceIdType.MESH,
            )
        pl.semaphore_wait(barrier, N_DEV - 1)

        rs_sends = []
        for kk in range(1, N_DEV):
            peer = jnp.mod(my + kk, N_DEV)
            rdma = pltpu.make_async_remote_copy(
                src_ref=t_ref.at[pl.ds(peer * ch, ch)],
                dst_ref=comm_ref.at[N_DEV - kk],
                send_sem=rs_ssem.at[kk],
                recv_sem=rs_rsem.at[N_DEV - kk],
                device_id=(peer,),
                device_id_type=pl.DeviceIdType.MESH,
            )
            rdma.start()
            rs_sends.append(rdma)

        acc = t_ref[pl.ds(my * ch, ch), :]
        for j in range(1, N_DEV):
            recv = pltpu.make_async_remote_copy(
                src_ref=comm_ref.at[j],
                dst_ref=comm_ref.at[j],
                send_sem=rs_ssem.at[j],
                recv_sem=rs_rsem.at[j],
                device_id=(my,),
                device_id_type=pl.DeviceIdType.MESH,
            )
            recv.wait_recv()
            acc = acc + comm_ref[j, :, :]

        y = jnp.dot(acc, w_ref[:, :], preferred_element_type=jnp.float32)
        out_ref[pl.ds(my * ch, ch), :] = y

        ag_sends = []
        for kk in range(1, N_DEV):
            peer = jnp.mod(my + kk, N_DEV)
            rdma = pltpu.make_async_remote_copy(
                src_ref=out_ref.at[pl.ds(my * ch, ch)],
                dst_ref=out_ref.at[pl.ds(my * ch, ch)],
                send_sem=ag_ssem.at[kk],
                recv_sem=ag_rsem.at[N_DEV - kk],
                device_id=(peer,),
                device_id_type=pl.DeviceIdType.MESH,
            )
            rdma.start()
            ag_sends.append(rdma)

        for rdma in rs_sends:
            rdma.wait_send()

        for j in range(1, N_DEV):
            src_dev = jnp.mod(my + j, N_DEV)
            recv = pltpu.make_async_remote_copy(
                src_ref=out_ref.at[pl.ds(src_dev * ch, ch)],
                dst_ref=out_ref.at[pl.ds(src_dev * ch, ch)],
                send_sem=ag_ssem.at[j],
                recv_sem=ag_rsem.at[j],
                device_id=(my,),
                device_id_type=pl.DeviceIdType.MESH,
            )
            recv.wait_recv()

        for rdma in ag_sends:
            rdma.wait_send()

    return pl.pallas_call(
        body,
        out_shape=jax.ShapeDtypeStruct((m, n), jnp.float32),
        in_specs=[
            pl.BlockSpec(memory_space=pltpu.VMEM),
            pl.BlockSpec(memory_space=pltpu.VMEM),
        ],
        out_specs=pl.BlockSpec(memory_space=pltpu.VMEM),
        scratch_shapes=[
            pltpu.VMEM((N_DEV, ch, k), jnp.float32),
            pltpu.SemaphoreType.DMA((N_DEV,)),
            pltpu.SemaphoreType.DMA((N_DEV,)),
            pltpu.SemaphoreType.DMA((N_DEV,)),
            pltpu.SemaphoreType.DMA((N_DEV,)),
        ],
        compiler_params=pltpu.CompilerParams(collective_id=0),
    )(t, W)
